# Initial kernel scaffold; baseline (speedup 1.0000x reference)
import functools

import jax
import jax.numpy as jnp
from jax import lax
from jax.experimental import pallas as pl
from jax.experimental.pallas import tpu as pltpu
from jax.experimental.pallas import tpu_sc as plsc

N = 10000
NP = 10240
E = 53333
CH = 128
EPT16 = 3584
EPAD = 16 * EPT16
ROWS = EPAD // CH
ZPT = NP // 16
OPT = N // 16

_sc_mesh = plsc.VectorSubcoreMesh(core_axis_name="c", subcore_axis_name="s")


@functools.partial(
    pl.kernel,
    out_type=jax.ShapeDtypeStruct((2, 3, NP, 16), jnp.float32),
    mesh=_sc_mesh,
    scratch_types=[
        pltpu.VMEM_SHARED((3, NP, 16), jnp.float32),
        pltpu.VMEM((28, CH), jnp.int32),
        pltpu.VMEM((CH, 16), jnp.float32),
        pltpu.SemaphoreType.DMA,
    ],
)
def _deg_kernel(esrc, edst, ones_hbm, zeros16, degs, deg_sh, idx2, ones_v, sem):
    core = lax.axis_index("c")
    tid = lax.axis_index("s")
    del sem
    pltpu.sync_copy(ones_hbm, ones_v)
    for r in range(3):
        pltpu.sync_copy(zeros16, deg_sh.at[r, pl.ds(tid * ZPT, ZPT)])
    plsc.subcore_barrier()
    for r in range(3):
        @pl.when(core == 0)
        def _():
            pltpu.sync_copy(esrc.at[r, pl.ds(tid * 28, 28)], idx2)

        @pl.when(core == 1)
        def _():
            pltpu.sync_copy(edst.at[r, pl.ds(tid * 28, 28)], idx2)

        for ch in range(28):
            pltpu.sync_copy(ones_v, deg_sh.at[r].at[idx2.at[ch]], add=True)
    plsc.subcore_barrier()
    for r in range(3):
        sl = pl.ds(tid * ZPT, ZPT)

        @pl.when(core == 0)
        def _():
            pltpu.sync_copy(deg_sh.at[r, sl], degs.at[0, r, sl])

        @pl.when(core == 1)
        def _():
            pltpu.sync_copy(deg_sh.at[r, sl], degs.at[1, r, sl])


@functools.partial(
    pl.kernel,
    out_type=[jax.ShapeDtypeStruct((N, 128), jnp.float32) for _ in range(6)],
    mesh=_sc_mesh,
    scratch_types=[
        pltpu.VMEM_SHARED((NP, 128), jnp.float32),
        pltpu.VMEM((28, CH), jnp.int32),
        pltpu.VMEM((28, CH), jnp.int32),
        pltpu.VMEM((CH, 128), jnp.float32),
        pltpu.SemaphoreType.DMA,
    ],
)
def _prop0_kernel(esrc, edst, zeros128, xs0, xs1, xs2, xs3, xs4, xs5,
                  a0, a1, a2, a3, a4, a5, agg_sh, sidx, didx, rows, sem):
    core = lax.axis_index("c")
    tid = lax.axis_index("s")
    xs = [xs0, xs1, xs2, xs3, xs4, xs5]
    outs = [a0, a1, a2, a3, a4, a5]
    for r in range(3):
        pltpu.sync_copy(zeros128, agg_sh.at[pl.ds(tid * ZPT, ZPT)])
        pltpu.sync_copy(esrc.at[r, pl.ds(tid * 28, 28)], sidx)
        pltpu.sync_copy(edst.at[r, pl.ds(tid * 28, 28)], didx)
        plsc.subcore_barrier()
        for ch in range(28):
            @pl.when(core == 0)
            def _(ch=ch, r=r):
                pltpu.async_copy(xs[2 * r].at[sidx.at[ch]], rows, sem).wait()

            @pl.when(core == 1)
            def _(ch=ch, r=r):
                pltpu.async_copy(xs[2 * r + 1].at[sidx.at[ch]], rows, sem).wait()

            pltpu.sync_copy(rows, agg_sh.at[didx.at[ch]], add=True)
        plsc.subcore_barrier()
        osl = pl.ds(tid * OPT, OPT)

        @pl.when(core == 0)
        def _(r=r):
            pltpu.sync_copy(agg_sh.at[osl], outs[2 * r].at[osl])

        @pl.when(core == 1)
        def _(r=r):
            pltpu.sync_copy(agg_sh.at[osl], outs[2 * r + 1].at[osl])

        plsc.subcore_barrier()


@functools.partial(
    pl.kernel,
    out_type=jax.ShapeDtypeStruct((3, 2, N, 64), jnp.float32),
    mesh=_sc_mesh,
    scratch_types=[
        pltpu.VMEM_SHARED((NP, 64), jnp.float32),
        pltpu.VMEM((14, CH), jnp.int32),
        pltpu.VMEM((14, CH), jnp.int32),
        pltpu.VMEM((CH, 64), jnp.float32),
        pltpu.SemaphoreType.DMA,
    ],
)
def _prop1_kernel(esrc, edst, zeros64, t0, t1, t2, part, agg_sh, sidx, didx,
                  rows, sem):
    core = lax.axis_index("c")
    tid = lax.axis_index("s")
    ts = [t0, t1, t2]
    for r in range(3):
        pltpu.sync_copy(zeros64, agg_sh.at[pl.ds(tid * ZPT, ZPT)])
        rbase = core * 224 + tid * 14
        pltpu.sync_copy(esrc.at[r, pl.ds(rbase, 14)], sidx)
        pltpu.sync_copy(edst.at[r, pl.ds(rbase, 14)], didx)
        plsc.subcore_barrier()
        for ch in range(14):
            pltpu.async_copy(ts[r].at[sidx.at[ch]], rows, sem).wait()
            pltpu.sync_copy(rows, agg_sh.at[didx.at[ch]], add=True)
        plsc.subcore_barrier()
        osl = pl.ds(tid * OPT, OPT)

        @pl.when(core == 0)
        def _(r=r):
            pltpu.sync_copy(agg_sh.at[osl], part.at[r, 0, osl])

        @pl.when(core == 1)
        def _(r=r):
            pltpu.sync_copy(agg_sh.at[osl], part.at[r, 1, osl])

        plsc.subcore_barrier()


def _norm_body(deg_ref, x_ref, xs0, xs1, xs2, xs3, xs4, xs5, ns_ref, nd_ref):
    deg = deg_ref[...]
    x = x_ref[...]
    xs = [xs0, xs1, xs2, xs3, xs4, xs5]
    ns_l, nd_l = [], []
    for r in range(3):
        ns = lax.rsqrt(jnp.maximum(deg[0, r, :, 0], 1.0))
        nd = lax.rsqrt(jnp.maximum(deg[1, r, :, 0], 1.0))
        ns_l.append(ns)
        nd_l.append(nd)
        for f in range(2):
            xs[2 * r + f][...] = x[:, 128 * f:128 * (f + 1)] * ns[:, None]
    ns_ref[...] = jnp.stack(ns_l)[:, :, None]
    nd_ref[...] = jnp.stack(nd_l)[:, :, None]


def _dense_body(a0, a1, a2, a3, a4, a5, ns_ref, nd_ref, w00, w01, w02,
                b0_ref, w1_ref, t0, t1, t2):
    aggs = [a0[...], a1[...], a2[...], a3[...], a4[...], a5[...]]
    ns = ns_ref[...]
    nd = nd_ref[...]
    b0 = b0_ref[...]
    w1 = w1_ref[...]
    w0 = [w00[...], w01[...], w02[...]]
    hacc = jnp.zeros((aggs[0].shape[0], 512), jnp.float32)
    for r in range(3):
        cat = jnp.concatenate([aggs[2 * r], aggs[2 * r + 1]], axis=1) * nd[r]
        y = jnp.dot(cat, w0[r], preferred_element_type=jnp.float32)
        hacc = hacc + jnp.maximum(y + b0[r][None, :], 0.0)
    h = hacc * (1.0 / 3.0)
    touts = [t0, t1, t2]
    for r in range(3):
        touts[r][...] = jnp.dot(h * ns[r], w1[r],
                                preferred_element_type=jnp.float32)


def _final_body(p_ref, nd_ref, b1_ref, out_ref):
    p = p_ref[...]
    nd = nd_ref[...]
    b1 = b1_ref[...]
    acc = jnp.zeros((p.shape[2], 64), jnp.float32)
    for r in range(3):
        acc = acc + (p[r, 0] + p[r, 1]) * nd[r]
    out_ref[...] = acc * (1.0 / 3.0) + (b1[0] + b1[1] + b1[2])[None, :] * (1.0 / 3.0)


def _pad_edges(e, pad_val):
    pad = jnp.full((EPAD - E,), pad_val, jnp.int32)
    return jnp.concatenate([e, pad]).reshape(ROWS, CH)


def kernel(x, edge_index_0, edge_index_1, edge_index_2,
           W0_0, b0_0, W0_1, b0_1, W0_2, b0_2,
           W1_0, b1_0, W1_1, b1_1, W1_2, b1_2):
    edges = [edge_index_0, edge_index_1, edge_index_2]
    esrc_h = jnp.stack([_pad_edges(e[0], N) for e in edges])
    esrc_g = jnp.stack([_pad_edges(e[0], 0) for e in edges])
    edst = jnp.stack([_pad_edges(e[1], N) for e in edges])
    ones16 = jnp.ones((CH, 16), jnp.float32)
    zeros16 = jnp.zeros((ZPT, 16), jnp.float32)
    zeros128 = jnp.zeros((ZPT, 128), jnp.float32)
    zeros64 = jnp.zeros((ZPT, 64), jnp.float32)

    degs = _deg_kernel(esrc_h, edst, ones16, zeros16)

    grid = (10,)
    B = 1000
    full2 = lambda s: pl.BlockSpec(s, lambda i: (0, 0))
    norm_out = (
        [jax.ShapeDtypeStruct((N, 128), jnp.float32) for _ in range(6)]
        + [jax.ShapeDtypeStruct((3, N, 1), jnp.float32) for _ in range(2)]
    )
    xs_spec = pl.BlockSpec((B, 128), lambda i: (i, 0))
    nvec_spec = pl.BlockSpec((3, B, 1), lambda i: (0, i, 0))
    res = pl.pallas_call(
        _norm_body,
        grid=grid,
        in_specs=[
            pl.BlockSpec((2, 3, B, 16), lambda i: (0, 0, i, 0)),
            pl.BlockSpec((B, 256), lambda i: (i, 0)),
        ],
        out_specs=[xs_spec] * 6 + [nvec_spec] * 2,
        out_shape=norm_out,
    )(degs, x)
    xs = res[:6]
    ns, nd = res[6], res[7]

    aggs = _prop0_kernel(esrc_g, edst, zeros128, *xs)

    b0s = jnp.stack([b0_0, b0_1, b0_2])
    w1s = jnp.stack([W1_0, W1_1, W1_2])
    ts = pl.pallas_call(
        _dense_body,
        grid=grid,
        in_specs=(
            [pl.BlockSpec((B, 128), lambda i: (i, 0))] * 6
            + [nvec_spec] * 2
            + [full2((256, 512))] * 3
            + [full2((3, 512)), pl.BlockSpec((3, 512, 64), lambda i: (0, 0, 0))]
        ),
        out_specs=[pl.BlockSpec((B, 64), lambda i: (i, 0))] * 3,
        out_shape=[jax.ShapeDtypeStruct((N, 64), jnp.float32)] * 3,
    )(*aggs, ns, nd, W0_0, W0_1, W0_2, b0s, w1s)

    part = _prop1_kernel(esrc_g, edst, zeros64, *ts)

    b1s = jnp.stack([b1_0, b1_1, b1_2])
    out = pl.pallas_call(
        _final_body,
        grid=grid,
        in_specs=[
            pl.BlockSpec((3, 2, B, 64), lambda i: (0, 0, i, 0)),
            nvec_spec,
            full2((3, 64)),
        ],
        out_specs=pl.BlockSpec((B, 64), lambda i: (i, 0)),
        out_shape=jax.ShapeDtypeStruct((N, 64), jnp.float32),
    )(part, nd, b1s)
    return out


# speedup vs baseline: 3.5202x; 3.5202x over previous
import functools

import jax
import jax.numpy as jnp
from jax import lax
from jax.experimental import pallas as pl
from jax.experimental.pallas import tpu as pltpu
from jax.experimental.pallas import tpu_sc as plsc

N = 10000
NP = 10240
E = 53333
CH = 128
EPT16 = 3584
EPAD = 16 * EPT16
ZPT = NP // 16

_sc_mesh = plsc.VectorSubcoreMesh(core_axis_name="c", subcore_axis_name="s")


@functools.partial(
    pl.kernel,
    out_type=jax.ShapeDtypeStruct((2, 3, NP, 128), jnp.float32),
    mesh=_sc_mesh,
    scratch_types=[
        pltpu.VMEM_SHARED((NP, 128), jnp.float32),
        pltpu.VMEM((28, CH), jnp.int32),
        pltpu.VMEM((CH, 128), jnp.float32),
        pltpu.SemaphoreType.DMA,
    ],
)
def _deg_kernel(esrc, edst, ones_hbm, zeros128, degs, deg_sh, idx2, ones_v, sem):
    core = lax.axis_index("c")
    tid = lax.axis_index("s")
    del sem
    pltpu.sync_copy(ones_hbm, ones_v)
    for r in range(3):
        pltpu.sync_copy(zeros128, deg_sh.at[pl.ds(tid * ZPT, ZPT)])

        @pl.when(core == 0)
        def _(r=r):
            pltpu.sync_copy(esrc.at[r, tid], idx2)

        @pl.when(core == 1)
        def _(r=r):
            pltpu.sync_copy(edst.at[r, tid], idx2)

        plsc.subcore_barrier()
        for ch in range(28):
            pltpu.sync_copy(ones_v, deg_sh.at[idx2.at[ch]], add=True)
        plsc.subcore_barrier()
        osl = pl.ds(tid * ZPT, ZPT)

        @pl.when(core == 0)
        def _(r=r):
            pltpu.sync_copy(deg_sh.at[osl], degs.at[0, r, osl])

        @pl.when(core == 1)
        def _(r=r):
            pltpu.sync_copy(deg_sh.at[osl], degs.at[1, r, osl])

        plsc.subcore_barrier()


@functools.partial(
    pl.kernel,
    out_type=[jax.ShapeDtypeStruct((NP, 128), jnp.float32) for _ in range(6)],
    mesh=_sc_mesh,
    scratch_types=[
        pltpu.VMEM_SHARED((NP, 128), jnp.float32),
        pltpu.VMEM((28, CH), jnp.int32),
        pltpu.VMEM((28, CH), jnp.int32),
        pltpu.VMEM((CH, 128), jnp.float32),
        pltpu.SemaphoreType.DMA,
    ],
)
def _prop0_kernel(esrc, edst, zeros128, xs0, xs1, xs2, xs3, xs4, xs5,
                  a0, a1, a2, a3, a4, a5, agg_sh, sidx, didx, rows, sem):
    core = lax.axis_index("c")
    tid = lax.axis_index("s")
    xs = [xs0, xs1, xs2, xs3, xs4, xs5]
    outs = [a0, a1, a2, a3, a4, a5]
    for r in range(3):
        pltpu.sync_copy(zeros128, agg_sh.at[pl.ds(tid * ZPT, ZPT)])
        pltpu.sync_copy(esrc.at[r, tid], sidx)
        pltpu.sync_copy(edst.at[r, tid], didx)
        plsc.subcore_barrier()
        for ch in range(28):
            @pl.when(core == 0)
            def _(ch=ch, r=r):
                pltpu.async_copy(xs[2 * r].at[sidx.at[ch]], rows, sem).wait()

            @pl.when(core == 1)
            def _(ch=ch, r=r):
                pltpu.async_copy(xs[2 * r + 1].at[sidx.at[ch]], rows, sem).wait()

            pltpu.sync_copy(rows, agg_sh.at[didx.at[ch]], add=True)
        plsc.subcore_barrier()
        osl = pl.ds(tid * ZPT, ZPT)

        @pl.when(core == 0)
        def _(r=r):
            pltpu.sync_copy(agg_sh.at[osl], outs[2 * r].at[osl])

        @pl.when(core == 1)
        def _(r=r):
            pltpu.sync_copy(agg_sh.at[osl], outs[2 * r + 1].at[osl])

        plsc.subcore_barrier()


@functools.partial(
    pl.kernel,
    out_type=jax.ShapeDtypeStruct((3, 2, NP, 128), jnp.float32),
    mesh=_sc_mesh,
    scratch_types=[
        pltpu.VMEM_SHARED((NP, 128), jnp.float32),
        pltpu.VMEM((14, CH), jnp.int32),
        pltpu.VMEM((14, CH), jnp.int32),
        pltpu.VMEM((CH, 128), jnp.float32),
        pltpu.SemaphoreType.DMA,
    ],
)
def _prop1_kernel(esrc, edst, zeros128, t0, t1, t2, part, agg_sh, sidx, didx,
                  rows, sem):
    core = lax.axis_index("c")
    tid = lax.axis_index("s")
    ts = [t0, t1, t2]
    for r in range(3):
        pltpu.sync_copy(zeros128, agg_sh.at[pl.ds(tid * ZPT, ZPT)])
        wid = core * 16 + tid
        pltpu.sync_copy(esrc.at[r, wid], sidx)
        pltpu.sync_copy(edst.at[r, wid], didx)
        plsc.subcore_barrier()
        for ch in range(14):
            pltpu.async_copy(ts[r].at[sidx.at[ch]], rows, sem).wait()
            pltpu.sync_copy(rows, agg_sh.at[didx.at[ch]], add=True)
        plsc.subcore_barrier()
        osl = pl.ds(tid * ZPT, ZPT)

        @pl.when(core == 0)
        def _(r=r):
            pltpu.sync_copy(agg_sh.at[osl], part.at[r, 0, osl])

        @pl.when(core == 1)
        def _(r=r):
            pltpu.sync_copy(agg_sh.at[osl], part.at[r, 1, osl])

        plsc.subcore_barrier()


def _norm_body(deg_ref, x_ref, xs0, xs1, xs2, xs3, xs4, xs5, ns_ref, nd_ref):
    deg = deg_ref[...]
    x = x_ref[...]
    xs = [xs0, xs1, xs2, xs3, xs4, xs5]
    ns_l, nd_l = [], []
    for r in range(3):
        ns = lax.rsqrt(jnp.maximum(deg[0, r, :, 0], 1.0))
        nd = lax.rsqrt(jnp.maximum(deg[1, r, :, 0], 1.0))
        ns_l.append(ns)
        nd_l.append(nd)
        for f in range(2):
            xs[2 * r + f][...] = x[:, 128 * f:128 * (f + 1)] * ns[:, None]
    ns_ref[...] = jnp.stack(ns_l)[:, :, None]
    nd_ref[...] = jnp.stack(nd_l)[:, :, None]


def _dense_body(a0, a1, a2, a3, a4, a5, ns_ref, nd_ref, w00, w01, w02,
                b0_ref, w1_ref, t0, t1, t2):
    aggs = [a0[...], a1[...], a2[...], a3[...], a4[...], a5[...]]
    ns = ns_ref[...]
    nd = nd_ref[...]
    b0 = b0_ref[...]
    w1 = w1_ref[...]
    w0 = [w00[...], w01[...], w02[...]]
    hacc = jnp.zeros((aggs[0].shape[0], 512), jnp.float32)
    for r in range(3):
        cat = jnp.concatenate([aggs[2 * r], aggs[2 * r + 1]], axis=1) * nd[r]
        y = jnp.dot(cat, w0[r], preferred_element_type=jnp.float32)
        hacc = hacc + jnp.maximum(y + b0[r][None, :], 0.0)
    h = hacc * (1.0 / 3.0)
    touts = [t0, t1, t2]
    zpad = jnp.zeros((h.shape[0], 64), jnp.float32)
    for r in range(3):
        t = jnp.dot(h * ns[r], w1[r], preferred_element_type=jnp.float32)
        touts[r][...] = jnp.concatenate([t, zpad], axis=1)


def _final_body(p_ref, nd_ref, b1_ref, out_ref):
    p = p_ref[...]
    nd = nd_ref[...]
    b1 = b1_ref[...]
    acc = jnp.zeros((p.shape[2], 64), jnp.float32)
    for r in range(3):
        acc = acc + (p[r, 0, :, :64] + p[r, 1, :, :64]) * nd[r]
    out_ref[...] = acc * (1.0 / 3.0) + (b1[0] + b1[1] + b1[2])[None, :] * (1.0 / 3.0)


def _pad_edges(e, pad_val):
    pad = jnp.full((EPAD - E,), pad_val, jnp.int32)
    return jnp.concatenate([e, pad])


def kernel(x, edge_index_0, edge_index_1, edge_index_2,
           W0_0, b0_0, W0_1, b0_1, W0_2, b0_2,
           W1_0, b1_0, W1_1, b1_1, W1_2, b1_2):
    edges = [edge_index_0, edge_index_1, edge_index_2]
    esrc_h = jnp.stack([_pad_edges(e[0], N) for e in edges]).reshape(3, 16, 28, CH)
    esrc_g16 = jnp.stack([_pad_edges(e[0], 0) for e in edges]).reshape(3, 16, 28, CH)
    edst16 = jnp.stack([_pad_edges(e[1], N) for e in edges]).reshape(3, 16, 28, CH)
    esrc_g32 = esrc_g16.reshape(3, 32, 14, CH)
    edst32 = edst16.reshape(3, 32, 14, CH)
    ones128 = jnp.ones((CH, 128), jnp.float32)
    zeros128 = jnp.zeros((ZPT, 128), jnp.float32)

    degs = _deg_kernel(esrc_h, edst16, ones128, zeros128)

    grid = (10,)
    B = 1000
    full2 = lambda s: pl.BlockSpec(s, lambda i: (0, 0))
    norm_out = (
        [jax.ShapeDtypeStruct((N, 128), jnp.float32) for _ in range(6)]
        + [jax.ShapeDtypeStruct((3, N, 1), jnp.float32) for _ in range(2)]
    )
    xs_spec = pl.BlockSpec((B, 128), lambda i: (i, 0))
    nvec_spec = pl.BlockSpec((3, B, 1), lambda i: (0, i, 0))
    res = pl.pallas_call(
        _norm_body,
        grid=grid,
        in_specs=[
            pl.BlockSpec((2, 3, B, 128), lambda i: (0, 0, i, 0)),
            pl.BlockSpec((B, 256), lambda i: (i, 0)),
        ],
        out_specs=[xs_spec] * 6 + [nvec_spec] * 2,
        out_shape=norm_out,
    )(degs, x)
    xs = res[:6]
    ns, nd = res[6], res[7]

    aggs = _prop0_kernel(esrc_g16, edst16, zeros128, *xs)

    b0s = jnp.stack([b0_0, b0_1, b0_2])
    w1s = jnp.stack([W1_0, W1_1, W1_2])
    ts = pl.pallas_call(
        _dense_body,
        grid=grid,
        in_specs=(
            [pl.BlockSpec((B, 128), lambda i: (i, 0))] * 6
            + [nvec_spec] * 2
            + [full2((256, 512))] * 3
            + [full2((3, 512)), pl.BlockSpec((3, 512, 64), lambda i: (0, 0, 0))]
        ),
        out_specs=[pl.BlockSpec((B, 128), lambda i: (i, 0))] * 3,
        out_shape=[jax.ShapeDtypeStruct((N, 128), jnp.float32)] * 3,
    )(*aggs, ns, nd, W0_0, W0_1, W0_2, b0s, w1s)

    part = _prop1_kernel(esrc_g32, edst32, zeros128, *ts)

    b1s = jnp.stack([b1_0, b1_1, b1_2])
    out = pl.pallas_call(
        _final_body,
        grid=grid,
        in_specs=[
            pl.BlockSpec((3, 2, B, 128), lambda i: (0, 0, i, 0)),
            nvec_spec,
            full2((3, 64)),
        ],
        out_specs=pl.BlockSpec((B, 64), lambda i: (i, 0)),
        out_shape=jax.ShapeDtypeStruct((N, 64), jnp.float32),
    )(part, nd, b1s)
    return out


# speedup vs baseline: 3.6526x; 1.0376x over previous
import functools

import jax
import jax.numpy as jnp
from jax import lax
from jax.experimental import pallas as pl
from jax.experimental.pallas import tpu as pltpu
from jax.experimental.pallas import tpu_sc as plsc

N = 10000
NP = 10240
E = 53333
CH = 128
EPT16 = 3584
EPAD = 16 * EPT16
ZPT = NP // 16

_sc_mesh = plsc.VectorSubcoreMesh(core_axis_name="c", subcore_axis_name="s")


@functools.partial(
    pl.kernel,
    out_type=jax.ShapeDtypeStruct((2, 3, NP, 128), jnp.float32),
    mesh=_sc_mesh,
    scratch_types=[
        pltpu.VMEM_SHARED((NP, 128), jnp.float32),
        pltpu.VMEM((28, CH), jnp.int32),
        pltpu.VMEM((CH, 128), jnp.float32),
        pltpu.SemaphoreType.DMA,
    ],
)
def _deg_kernel(esrc, edst, ones_hbm, zeros128, degs, deg_sh, idx2, ones_v, sem):
    core = lax.axis_index("c")
    tid = lax.axis_index("s")
    del sem
    pltpu.sync_copy(ones_hbm, ones_v)
    for r in range(3):
        pltpu.sync_copy(zeros128, deg_sh.at[pl.ds(tid * ZPT, ZPT)])

        @pl.when(core == 0)
        def _(r=r):
            pltpu.sync_copy(esrc.at[r, tid], idx2)

        @pl.when(core == 1)
        def _(r=r):
            pltpu.sync_copy(edst.at[r, tid], idx2)

        plsc.subcore_barrier()
        for ch in range(28):
            pltpu.sync_copy(ones_v, deg_sh.at[idx2.at[ch]], add=True)
        plsc.subcore_barrier()
        osl = pl.ds(tid * ZPT, ZPT)

        @pl.when(core == 0)
        def _(r=r):
            pltpu.sync_copy(deg_sh.at[osl], degs.at[0, r, osl])

        @pl.when(core == 1)
        def _(r=r):
            pltpu.sync_copy(deg_sh.at[osl], degs.at[1, r, osl])

        plsc.subcore_barrier()


@functools.partial(
    pl.kernel,
    out_type=[jax.ShapeDtypeStruct((NP, 128), jnp.float32) for _ in range(6)],
    mesh=_sc_mesh,
    scratch_types=[
        pltpu.VMEM_SHARED((NP, 128), jnp.float32),
        pltpu.VMEM((28, CH), jnp.int32),
        pltpu.VMEM((28, CH), jnp.int32),
        pltpu.VMEM((CH, 128), jnp.float32),
        pltpu.VMEM((CH, 128), jnp.float32),
        pltpu.SemaphoreType.DMA,
        pltpu.SemaphoreType.DMA,
    ],
)
def _prop0_kernel(esrc, edst, zeros128, xs0, xs1, xs2, xs3, xs4, xs5,
                  a0, a1, a2, a3, a4, a5, agg_sh, sidx, didx, rows0, rows1,
                  sem0, sem1):
    core = lax.axis_index("c")
    tid = lax.axis_index("s")
    xs = [xs0, xs1, xs2, xs3, xs4, xs5]
    outs = [a0, a1, a2, a3, a4, a5]
    bufs = [rows0, rows1]
    sems = [sem0, sem1]

    def chunkloop(table, nch):
        pend = pltpu.async_copy(table.at[sidx.at[0]], bufs[0], sems[0])
        for ch in range(nch):
            cur, csem = bufs[ch % 2], sems[ch % 2]
            if ch + 1 < nch:
                nxt = pltpu.async_copy(table.at[sidx.at[ch + 1]],
                                       bufs[(ch + 1) % 2], sems[(ch + 1) % 2])
            pend.wait()
            if ch + 1 < nch:
                pend = nxt
            pltpu.sync_copy(cur, agg_sh.at[didx.at[ch]], add=True)

    for r in range(3):
        pltpu.sync_copy(zeros128, agg_sh.at[pl.ds(tid * ZPT, ZPT)])
        pltpu.sync_copy(esrc.at[r, tid], sidx)
        pltpu.sync_copy(edst.at[r, tid], didx)
        plsc.subcore_barrier()

        @pl.when(core == 0)
        def _(r=r):
            chunkloop(xs[2 * r], 28)

        @pl.when(core == 1)
        def _(r=r):
            chunkloop(xs[2 * r + 1], 28)

        plsc.subcore_barrier()
        osl = pl.ds(tid * ZPT, ZPT)

        @pl.when(core == 0)
        def _(r=r):
            pltpu.sync_copy(agg_sh.at[osl], outs[2 * r].at[osl])

        @pl.when(core == 1)
        def _(r=r):
            pltpu.sync_copy(agg_sh.at[osl], outs[2 * r + 1].at[osl])

        plsc.subcore_barrier()


@functools.partial(
    pl.kernel,
    out_type=jax.ShapeDtypeStruct((3, 2, NP, 128), jnp.float32),
    mesh=_sc_mesh,
    scratch_types=[
        pltpu.VMEM_SHARED((NP, 128), jnp.float32),
        pltpu.VMEM((14, CH), jnp.int32),
        pltpu.VMEM((14, CH), jnp.int32),
        pltpu.VMEM((CH, 128), jnp.float32),
        pltpu.VMEM((CH, 128), jnp.float32),
        pltpu.SemaphoreType.DMA,
        pltpu.SemaphoreType.DMA,
    ],
)
def _prop1_kernel(esrc, edst, zeros128, t0, t1, t2, part, agg_sh, sidx, didx,
                  rows0, rows1, sem0, sem1):
    core = lax.axis_index("c")
    tid = lax.axis_index("s")
    ts = [t0, t1, t2]
    for r in range(3):
        pltpu.sync_copy(zeros128, agg_sh.at[pl.ds(tid * ZPT, ZPT)])
        wid = core * 16 + tid
        pltpu.sync_copy(esrc.at[r, wid], sidx)
        pltpu.sync_copy(edst.at[r, wid], didx)
        plsc.subcore_barrier()
        bufs = [rows0, rows1]
        sems = [sem0, sem1]
        pend = pltpu.async_copy(ts[r].at[sidx.at[0]], bufs[0], sems[0])
        for ch in range(14):
            cur = bufs[ch % 2]
            if ch + 1 < 14:
                nxt = pltpu.async_copy(ts[r].at[sidx.at[ch + 1]],
                                       bufs[(ch + 1) % 2], sems[(ch + 1) % 2])
            pend.wait()
            if ch + 1 < 14:
                pend = nxt
            pltpu.sync_copy(cur, agg_sh.at[didx.at[ch]], add=True)
        plsc.subcore_barrier()
        osl = pl.ds(tid * ZPT, ZPT)

        @pl.when(core == 0)
        def _(r=r):
            pltpu.sync_copy(agg_sh.at[osl], part.at[r, 0, osl])

        @pl.when(core == 1)
        def _(r=r):
            pltpu.sync_copy(agg_sh.at[osl], part.at[r, 1, osl])

        plsc.subcore_barrier()


def _norm_body(deg_ref, x_ref, xs0, xs1, xs2, xs3, xs4, xs5, ns_ref, nd_ref):
    deg = deg_ref[...]
    x = x_ref[...]
    xs = [xs0, xs1, xs2, xs3, xs4, xs5]
    ns_l, nd_l = [], []
    for r in range(3):
        ns = lax.rsqrt(jnp.maximum(deg[0, r, :, 0], 1.0))
        nd = lax.rsqrt(jnp.maximum(deg[1, r, :, 0], 1.0))
        ns_l.append(ns)
        nd_l.append(nd)
        for f in range(2):
            xs[2 * r + f][...] = x[:, 128 * f:128 * (f + 1)] * ns[:, None]
    ns_ref[...] = jnp.stack(ns_l)[:, :, None]
    nd_ref[...] = jnp.stack(nd_l)[:, :, None]


def _dense_body(a0, a1, a2, a3, a4, a5, ns_ref, nd_ref, w00, w01, w02,
                b0_ref, w1_ref, t0, t1, t2):
    aggs = [a0[...], a1[...], a2[...], a3[...], a4[...], a5[...]]
    ns = ns_ref[...]
    nd = nd_ref[...]
    b0 = b0_ref[...]
    w1 = w1_ref[...]
    w0 = [w00[...], w01[...], w02[...]]
    hacc = jnp.zeros((aggs[0].shape[0], 512), jnp.float32)
    for r in range(3):
        cat = jnp.concatenate([aggs[2 * r], aggs[2 * r + 1]], axis=1) * nd[r]
        y = jnp.dot(cat, w0[r], preferred_element_type=jnp.float32)
        hacc = hacc + jnp.maximum(y + b0[r][None, :], 0.0)
    h = hacc * (1.0 / 3.0)
    touts = [t0, t1, t2]
    zpad = jnp.zeros((h.shape[0], 64), jnp.float32)
    for r in range(3):
        t = jnp.dot(h * ns[r], w1[r], preferred_element_type=jnp.float32)
        touts[r][...] = jnp.concatenate([t, zpad], axis=1)


def _final_body(p_ref, nd_ref, b1_ref, out_ref):
    p = p_ref[...]
    nd = nd_ref[...]
    b1 = b1_ref[...]
    acc = jnp.zeros((p.shape[2], 64), jnp.float32)
    for r in range(3):
        acc = acc + (p[r, 0, :, :64] + p[r, 1, :, :64]) * nd[r]
    out_ref[...] = acc * (1.0 / 3.0) + (b1[0] + b1[1] + b1[2])[None, :] * (1.0 / 3.0)


def _pad_edges(e, pad_val):
    pad = jnp.full((EPAD - E,), pad_val, jnp.int32)
    return jnp.concatenate([e, pad])


def kernel(x, edge_index_0, edge_index_1, edge_index_2,
           W0_0, b0_0, W0_1, b0_1, W0_2, b0_2,
           W1_0, b1_0, W1_1, b1_1, W1_2, b1_2):
    edges = [edge_index_0, edge_index_1, edge_index_2]
    esrc_h = jnp.stack([_pad_edges(e[0], N) for e in edges]).reshape(3, 16, 28, CH)
    esrc_g16 = jnp.stack([_pad_edges(e[0], 0) for e in edges]).reshape(3, 16, 28, CH)
    edst16 = jnp.stack([_pad_edges(e[1], N) for e in edges]).reshape(3, 16, 28, CH)
    esrc_g32 = esrc_g16.reshape(3, 32, 14, CH)
    edst32 = edst16.reshape(3, 32, 14, CH)
    ones128 = jnp.ones((CH, 128), jnp.float32)
    zeros128 = jnp.zeros((ZPT, 128), jnp.float32)

    degs = _deg_kernel(esrc_h, edst16, ones128, zeros128)

    grid = (10,)
    B = 1000
    full2 = lambda s: pl.BlockSpec(s, lambda i: (0, 0))
    norm_out = (
        [jax.ShapeDtypeStruct((N, 128), jnp.float32) for _ in range(6)]
        + [jax.ShapeDtypeStruct((3, N, 1), jnp.float32) for _ in range(2)]
    )
    xs_spec = pl.BlockSpec((B, 128), lambda i: (i, 0))
    nvec_spec = pl.BlockSpec((3, B, 1), lambda i: (0, i, 0))
    res = pl.pallas_call(
        _norm_body,
        grid=grid,
        in_specs=[
            pl.BlockSpec((2, 3, B, 128), lambda i: (0, 0, i, 0)),
            pl.BlockSpec((B, 256), lambda i: (i, 0)),
        ],
        out_specs=[xs_spec] * 6 + [nvec_spec] * 2,
        out_shape=norm_out,
    )(degs, x)
    xs = res[:6]
    ns, nd = res[6], res[7]

    aggs = _prop0_kernel(esrc_g16, edst16, zeros128, *xs)

    b0s = jnp.stack([b0_0, b0_1, b0_2])
    w1s = jnp.stack([W1_0, W1_1, W1_2])
    ts = pl.pallas_call(
        _dense_body,
        grid=grid,
        in_specs=(
            [pl.BlockSpec((B, 128), lambda i: (i, 0))] * 6
            + [nvec_spec] * 2
            + [full2((256, 512))] * 3
            + [full2((3, 512)), pl.BlockSpec((3, 512, 64), lambda i: (0, 0, 0))]
        ),
        out_specs=[pl.BlockSpec((B, 128), lambda i: (i, 0))] * 3,
        out_shape=[jax.ShapeDtypeStruct((N, 128), jnp.float32)] * 3,
    )(*aggs, ns, nd, W0_0, W0_1, W0_2, b0s, w1s)

    part = _prop1_kernel(esrc_g32, edst32, zeros128, *ts)

    b1s = jnp.stack([b1_0, b1_1, b1_2])
    out = pl.pallas_call(
        _final_body,
        grid=grid,
        in_specs=[
            pl.BlockSpec((3, 2, B, 128), lambda i: (0, 0, i, 0)),
            nvec_spec,
            full2((3, 64)),
        ],
        out_specs=pl.BlockSpec((B, 64), lambda i: (i, 0)),
        out_shape=jax.ShapeDtypeStruct((N, 64), jnp.float32),
    )(part, nd, b1s)
    return out
